# baseline (speedup 1.0000x reference)
import functools
import math

import jax
import jax.numpy as jnp
from jax import lax
from jax.experimental import pallas as pl
from jax.experimental.pallas import tpu as pltpu
from jax.experimental.pallas import tpu_sc as plsc

N = 10000
E = 160000
HID = 128
HEADS = 8
DH = HID // HEADS
G = 50
FIL = 128
L = 2
CUTOFF = 10.0
SHIFT = float(math.log(2.0))
INV_SQRT_DH = 1.0 / math.sqrt(float(DH))

TILE_E = 1280
TILE_N = 1000

_PERM = []
for _h2 in range(HEADS // 2):
    for _d in range(DH):
        _PERM.append(32 * _h2 + _d)
        _PERM.append(32 * _h2 + DH + _d)

DEN_W = 16
BB = 32
NB_TOT = E // BB
NW = 32
NB_PER_W = -(-NB_TOT // NW)
N_PAD = 10112
ROWS_PER_TILE = N_PAD // 16


def _ssp(v):
    return jax.nn.softplus(v) - SHIFT


def _edge_body(ea_ref, ew_ref, w1_ref, b1_ref, w2_ref, b2_ref, we_ref, be_ref,
               out_ref):
    t = jnp.dot(ea_ref[...], w1_ref[...], preferred_element_type=jnp.float32)
    t = _ssp(t + b1_ref[...])
    e = jnp.dot(t, w2_ref[...], preferred_element_type=jnp.float32) + b2_ref[...]
    c = 0.5 * (jnp.cos(ew_ref[...] * (math.pi / CUTOFF)) + 1.0)
    e = e * c
    out_ref[...] = (jnp.dot(e, we_ref[...], preferred_element_type=jnp.float32)
                    + be_ref[...]).astype(jnp.bfloat16)


def _edge_proj(edge_attr, ew2d, w1, b1, w2, b2, we, be):
    grid = (E // TILE_E,)
    return pl.pallas_call(
        _edge_body,
        grid=grid,
        in_specs=[
            pl.BlockSpec((TILE_E, G), lambda i: (i, 0)),
            pl.BlockSpec((TILE_E, 1), lambda i: (i, 0)),
            pl.BlockSpec((G, FIL), lambda i: (0, 0)),
            pl.BlockSpec((1, FIL), lambda i: (0, 0)),
            pl.BlockSpec((FIL, FIL), lambda i: (0, 0)),
            pl.BlockSpec((1, FIL), lambda i: (0, 0)),
            pl.BlockSpec((FIL, HID), lambda i: (0, 0)),
            pl.BlockSpec((1, HID), lambda i: (0, 0)),
        ],
        out_specs=pl.BlockSpec((TILE_E, HID), lambda i: (i, 0)),
        out_shape=jax.ShapeDtypeStruct((E, HID), jnp.bfloat16),
    )(edge_attr, ew2d, w1, b1, w2, b2, we, be)


def _qkvs_body(h_ref, wq_ref, wk_ref, wv_ref, ws_ref, b_ref, q_ref, k_ref,
               v_ref, s_ref):
    h = h_ref[...]
    b = b_ref[...]
    q_ref[...] = (jnp.dot(h, wq_ref[...], preferred_element_type=jnp.float32)
                  + b[:, 0:HID]).astype(jnp.bfloat16)
    k_ref[...] = (jnp.dot(h, wk_ref[...], preferred_element_type=jnp.float32)
                  + b[:, HID:2 * HID]).astype(jnp.bfloat16)
    v_ref[...] = (jnp.dot(h, wv_ref[...], preferred_element_type=jnp.float32)
                  + b[:, 2 * HID:3 * HID]).astype(jnp.bfloat16)
    s_ref[...] = jnp.dot(h, ws_ref[...], preferred_element_type=jnp.float32) + b[:, 3 * HID:4 * HID]


def _qkvs_proj(h, wq, wk, wv, ws, bcat):
    grid = (N // TILE_N,)
    wspec = pl.BlockSpec((HID, HID), lambda i: (0, 0))
    nspec = pl.BlockSpec((TILE_N, HID), lambda i: (i, 0))
    return pl.pallas_call(
        _qkvs_body,
        grid=grid,
        in_specs=[nspec, wspec, wspec, wspec, wspec,
                  pl.BlockSpec((1, 4 * HID), lambda i: (0, 0))],
        out_specs=[nspec, nspec, nspec, nspec],
        out_shape=[jax.ShapeDtypeStruct((N, HID), jnp.bfloat16)] * 3
        + [jax.ShapeDtypeStruct((N, HID), jnp.float32)],
    )(h, wq, wk, wv, ws, bcat)


def _final_body(h_ref, w_ref, b_ref, out_ref):
    out_ref[...] = (jnp.dot(h_ref[...], w_ref[...],
                            preferred_element_type=jnp.float32) + b_ref[...])


def _final_proj(h, w, b2d):
    grid = (N // TILE_N,)
    return pl.pallas_call(
        _final_body,
        grid=grid,
        in_specs=[
            pl.BlockSpec((TILE_N, HID), lambda i: (i, 0)),
            pl.BlockSpec((HID, HID), lambda i: (0, 0)),
            pl.BlockSpec((1, HID), lambda i: (0, 0)),
        ],
        out_specs=pl.BlockSpec((TILE_N, HID), lambda i: (i, 0)),
        out_shape=jax.ShapeDtypeStruct((N, HID), jnp.float32),
    )(h, w, b2d)


def _post_body(pa_ref, pd_ref, rep_ref, skip_ref, h_ref, lw_ref, lb_ref,
               out_ref):
    pa = pa_ref[...]
    pd = pd_ref[...]
    aggnum = pa[0] + pa[1]
    den = pd[0, :, 0:HEADS] + pd[1, :, 0:HEADS]
    denrep = jnp.dot(den, rep_ref[...], preferred_element_type=jnp.float32)
    agg = aggnum / (denrep + 1e-16)
    t = _ssp(agg + skip_ref[...])
    out_ref[...] = (h_ref[...]
                    + jnp.dot(t, lw_ref[...], preferred_element_type=jnp.float32)
                    + lb_ref[...])


def _post(pagg, pden, rep, skip, h, lin_w, lin_b):
    grid = (N // TILE_N,)
    return pl.pallas_call(
        _post_body,
        grid=grid,
        in_specs=[
            pl.BlockSpec((2, TILE_N, HID), lambda i: (0, i, 0)),
            pl.BlockSpec((2, TILE_N, DEN_W), lambda i: (0, i, 0)),
            pl.BlockSpec((HEADS, HID), lambda i: (0, 0)),
            pl.BlockSpec((TILE_N, HID), lambda i: (i, 0)),
            pl.BlockSpec((TILE_N, HID), lambda i: (i, 0)),
            pl.BlockSpec((HID, HID), lambda i: (0, 0)),
            pl.BlockSpec((1, HID), lambda i: (0, 0)),
        ],
        out_specs=pl.BlockSpec((TILE_N, HID), lambda i: (i, 0)),
        out_shape=jax.ShapeDtypeStruct((N, HID), jnp.float32),
    )(pagg, pden, rep, skip, h, lin_w, lin_b)


def _attn_sc_body(q_hbm, k_hbm, v_hbm, ep_hbm, src_hbm, dst_hbm,
                  agg_hbm, den_hbm,
                  src_v0, src_v1, dst_v0, dst_v1, dst_s0, dst_s1,
                  q_r0, q_r1, k_r0, k_r1, v_r0, v_r1, e_r0, e_r1,
                  w_r0, w_r1, ea_b0, ea_b1,
                  agg_tbl, den_tbl,
                  sem_i0, sem_i1, sem_g0, sem_g1, sem_s0, sem_s1):
    cid = lax.axis_index("c")
    sid = lax.axis_index("s")
    w = cid * 16 + sid

    src_v = (src_v0, src_v1)
    dst_v = (dst_v0, dst_v1)
    dst_s = (dst_s0, dst_s1)
    q_r = (q_r0, q_r1)
    k_r = (k_r0, k_r1)
    v_r = (v_r0, v_r1)
    e_r = (e_r0, e_r1)
    w_r = (w_r0, w_r1)
    ea_b = (ea_b0, ea_b1)
    sem_i = (sem_i0, sem_i1)
    sem_g = (sem_g0, sem_g1)
    sem_s = (sem_s0, sem_s1)

    lanes = lax.iota(jnp.int32, 16)
    zf = jnp.zeros((16,), jnp.float32)

    def _zero_bufs(r, carry):
        rows = jnp.zeros((16,), jnp.int32) + r
        for c in range(HID // 16):
            plsc.store_scatter(w_r0, [rows, c * 16 + lanes], zf)
        plsc.store_scatter(ea_b0, [rows, lanes], zf)
        plsc.store_scatter(ea_b1, [rows, lanes], zf)
        return carry
    lax.fori_loop(0, BB, _zero_bufs, 0)

    r0 = sid * ROWS_PER_TILE
    for off in range(0, ROWS_PER_TILE, BB):
        n = min(BB, ROWS_PER_TILE - off)
        pltpu.sync_copy(w_r0.at[pl.ds(0, n)], agg_tbl.at[pl.ds(r0 + off, n)])
        pltpu.sync_copy(ea_b0.at[pl.ds(0, n)], den_tbl.at[pl.ds(r0 + off, n)])
    plsc.subcore_barrier()

    def _issue_idx(slot, p):
        base = (w + NW * slot) * BB
        pltpu.async_copy(src_hbm.at[pl.ds(base, BB)], src_v[p], sem_i[p])
        pltpu.async_copy(dst_hbm.at[pl.ds(base, BB)], dst_v[p], sem_i[p])

    def _wait_idx(slot, p):
        base = (w + NW * slot) * BB
        pltpu.make_async_copy(src_hbm.at[pl.ds(base, BB)], src_v[p],
                              sem_i[p]).wait()
        pltpu.make_async_copy(dst_hbm.at[pl.ds(base, BB)], dst_v[p],
                              sem_i[p]).wait()

    def _issue_gathers(slot, p):
        base = (w + NW * slot) * BB
        pltpu.async_copy(q_hbm.at[dst_v[p]], q_r[p], sem_g[p])
        pltpu.async_copy(k_hbm.at[src_v[p]], k_r[p], sem_g[p])
        pltpu.async_copy(v_hbm.at[src_v[p]], v_r[p], sem_g[p])
        pltpu.async_copy(ep_hbm.at[pl.ds(base, BB)], e_r[p], sem_g[p])

    def _wait_gathers(slot, p):
        base = (w + NW * slot) * BB
        pltpu.make_async_copy(q_hbm.at[dst_v[p]], q_r[p], sem_g[p]).wait()
        pltpu.make_async_copy(k_hbm.at[src_v[p]], k_r[p], sem_g[p]).wait()
        pltpu.make_async_copy(v_hbm.at[src_v[p]], v_r[p], sem_g[p]).wait()
        pltpu.make_async_copy(ep_hbm.at[pl.ds(base, BB)], e_r[p],
                              sem_g[p]).wait()

    def _copy_dst(p):
        for t in range(BB // 16):
            dst_s[p][pl.ds(t * 16, 16)] = dst_v[p][pl.ds(t * 16, 16)]

    def _issue_scatter(p):
        pltpu.async_copy(w_r[p], agg_tbl.at[dst_s[p]], sem_s[p], add=True)
        pltpu.async_copy(ea_b[p], den_tbl.at[dst_s[p]], sem_s[p], add=True)

    def _wait_scatter(p):
        pltpu.make_async_copy(w_r[p], agg_tbl.at[dst_s[p]], sem_s[p]).wait()
        pltpu.make_async_copy(ea_b[p], den_tbl.at[dst_s[p]], sem_s[p]).wait()

    def _compute(p):
        def _edge(e, carry2):
            earow = zf
            for h2 in range(HEADS // 2):
                sl = pl.ds(32 * h2, 32)
                fmt = plsc.PackFormat.INTERLEAVED
                qa, qb = plsc.unpack(q_r[p][e, sl], format=fmt)
                ka, kb = plsc.unpack(k_r[p][e, sl], format=fmt)
                ga, gb = plsc.unpack(e_r[p][e, sl], format=fmt)
                va, vb = plsc.unpack(v_r[p][e, sl], format=fmt)
                sa = jnp.sum(qa * (ka + ga))
                sb = jnp.sum(qb * (kb + gb))
                aa = jnp.exp(jnp.full((16,), sa, jnp.float32) * INV_SQRT_DH)
                ab = jnp.exp(jnp.full((16,), sb, jnp.float32) * INV_SQRT_DH)
                w_r[p][e, pl.ds(32 * h2, DH)] = (va + ga) * aa
                w_r[p][e, pl.ds(32 * h2 + DH, DH)] = (vb + gb) * ab
                earow = jnp.where(lanes == 2 * h2, aa, earow)
                earow = jnp.where(lanes == 2 * h2 + 1, ab, earow)
            ea_b[p][e, pl.ds(0, DEN_W)] = earow
            return carry2
        lax.fori_loop(0, BB, _edge, 0)

    def _valid(slot):
        return (w + NW * slot) < NB_TOT

    _issue_idx(0, 0)
    _wait_idx(0, 0)
    _issue_gathers(0, 0)
    _copy_dst(0)

    @pl.when(_valid(1))
    def _():
        _issue_idx(1, 1)

    def _slot(j, p):
        jn1 = j + 1
        jn2 = j + 2

        @pl.when(_valid(jn1))
        def _():
            _wait_idx(jn1, 1 - p)
            _issue_gathers(jn1, 1 - p)
            _copy_dst(1 - p)

        @pl.when(_valid(j))
        def _():
            _wait_gathers(j, p)

        @pl.when(_valid(jn2))
        def _():
            _issue_idx(jn2, p)

        @pl.when(_valid(j))
        def _():
            _compute(p)
            _issue_scatter(p)

    def _pair(jj, carry):
        j = 2 * jj

        @pl.when((j >= 1) & _valid(j - 1))
        def _():
            _wait_scatter(1)
        _slot(j, 0)

        @pl.when(_valid(j))
        def _():
            _wait_scatter(0)
        _slot(j + 1, 1)
        return carry
    lax.fori_loop(0, (NB_PER_W + 1) // 2, _pair, 0)

    plsc.subcore_barrier()
    for off in range(0, ROWS_PER_TILE, BB):
        n = min(BB, ROWS_PER_TILE - off)
        pltpu.sync_copy(agg_tbl.at[pl.ds(r0 + off, n)], w_r0.at[pl.ds(0, n)])
        pltpu.sync_copy(w_r0.at[pl.ds(0, n)],
                        agg_hbm.at[cid, pl.ds(r0 + off, n)])
        pltpu.sync_copy(den_tbl.at[pl.ds(r0 + off, n)], ea_b0.at[pl.ds(0, n)])
        pltpu.sync_copy(ea_b0.at[pl.ds(0, n)],
                        den_hbm.at[cid, pl.ds(r0 + off, n)])


@functools.partial(
    pl.kernel,
    mesh=plsc.VectorSubcoreMesh(core_axis_name="c", subcore_axis_name="s"),
    out_type=(jax.ShapeDtypeStruct((2, N_PAD, HID), jnp.float32),
              jax.ShapeDtypeStruct((2, N_PAD, DEN_W), jnp.float32)),
    compiler_params=pltpu.CompilerParams(use_tc_tiling_on_sc=False,
                                         needs_layout_passes=False),
    scratch_types=(
        [pltpu.VMEM((BB,), jnp.int32)] * 6
        + [pltpu.VMEM((BB, HID), jnp.bfloat16)] * 8
        + [pltpu.VMEM((BB, HID), jnp.float32)] * 2
        + [pltpu.VMEM((BB, DEN_W), jnp.float32)] * 2
        + [pltpu.VMEM_SHARED((N_PAD, HID), jnp.float32),
           pltpu.VMEM_SHARED((N_PAD, DEN_W), jnp.float32)]
        + [pltpu.SemaphoreType.DMA] * 6
    ),
)
def _attn_sc(q_hbm, k_hbm, v_hbm, ep_hbm, src_hbm, dst_hbm, agg_hbm, den_hbm,
             *rest):
    _attn_sc_body(q_hbm, k_hbm, v_hbm, ep_hbm, src_hbm, dst_hbm,
                  agg_hbm, den_hbm, *rest)


def kernel(x, edge_index, edge_weight, edge_attr, mlp_w1, mlp_b1, mlp_w2,
           mlp_b2, wq, bq, wk, bk, wv, bv, we, be, wskip, bskip, lin_w, lin_b,
           out_w, out_b):
    src = edge_index[0]
    dst = edge_index[1]
    ew2d = edge_weight.reshape(E, 1)
    rep = jnp.repeat(jnp.eye(HEADS, dtype=jnp.float32), DH, axis=1)
    perm = jnp.asarray(_PERM, dtype=jnp.int32)
    h = x
    for l in range(L):
        ep = _edge_proj(edge_attr, ew2d, mlp_w1[l], mlp_b1[l].reshape(1, FIL),
                        mlp_w2[l], mlp_b2[l].reshape(1, FIL), we[l][:, perm],
                        be[l][perm].reshape(1, HID))
        bcat = jnp.concatenate([bq[l][perm], bk[l][perm], bv[l][perm],
                                bskip[l]]).reshape(1, 4 * HID)
        q, k, v, skip = _qkvs_proj(h, wq[l][:, perm], wk[l][:, perm],
                                   wv[l][:, perm], wskip[l], bcat)
        pagg, pden = _attn_sc(q, k, v, ep, src, dst)
        h = _post(pagg, pden, rep, skip, h, lin_w[l], lin_b[l].reshape(1, HID))
    return _final_proj(h, out_w, out_b.reshape(1, HID))


# speedup vs baseline: 1.5805x; 1.5805x over previous
import functools
import math

import jax
import jax.numpy as jnp
from jax import lax
from jax.experimental import pallas as pl
from jax.experimental.pallas import tpu as pltpu
from jax.experimental.pallas import tpu_sc as plsc

N = 10000
E = 160000
HID = 128
HEADS = 8
DH = HID // HEADS
G = 50
FIL = 128
L = 2
CUTOFF = 10.0
SHIFT = float(math.log(2.0))
INV_SQRT_DH = 1.0 / math.sqrt(float(DH))

TILE_E = 1280
TILE_N = 1000

DEN_W = 16
BB = 32
NB_TOT = E // BB
NW = 32
NB_PER_W = -(-NB_TOT // NW)
N_PAD = 10112
ROWS_PER_TILE = N_PAD // 16


def _ssp(v):
    return jax.nn.softplus(v) - SHIFT


def _edge_body(ea_ref, ew_ref, w1_ref, b1_ref, w2_ref, b2_ref, we_ref, be_ref,
               out_ref):
    t = jnp.dot(ea_ref[...], w1_ref[...], preferred_element_type=jnp.float32)
    t = _ssp(t + b1_ref[...])
    e = jnp.dot(t, w2_ref[...], preferred_element_type=jnp.float32) + b2_ref[...]
    c = 0.5 * (jnp.cos(ew_ref[...] * (math.pi / CUTOFF)) + 1.0)
    e = e * c
    out_ref[...] = (jnp.dot(e, we_ref[...], preferred_element_type=jnp.float32)
                    + be_ref[...])


def _edge_proj(edge_attr, ew2d, w1, b1, w2, b2, we, be):
    grid = (E // TILE_E,)
    return pl.pallas_call(
        _edge_body,
        grid=grid,
        in_specs=[
            pl.BlockSpec((TILE_E, G), lambda i: (i, 0)),
            pl.BlockSpec((TILE_E, 1), lambda i: (i, 0)),
            pl.BlockSpec((G, FIL), lambda i: (0, 0)),
            pl.BlockSpec((1, FIL), lambda i: (0, 0)),
            pl.BlockSpec((FIL, FIL), lambda i: (0, 0)),
            pl.BlockSpec((1, FIL), lambda i: (0, 0)),
            pl.BlockSpec((FIL, HID), lambda i: (0, 0)),
            pl.BlockSpec((1, HID), lambda i: (0, 0)),
        ],
        out_specs=pl.BlockSpec((TILE_E, HID), lambda i: (i, 0)),
        out_shape=jax.ShapeDtypeStruct((E, HID), jnp.float32),
    )(edge_attr, ew2d, w1, b1, w2, b2, we, be)


def _qkvs_body(h_ref, wq_ref, wk_ref, wv_ref, ws_ref, b_ref, q_ref, k_ref,
               v_ref, s_ref):
    h = h_ref[...]
    b = b_ref[...]
    q_ref[...] = jnp.dot(h, wq_ref[...], preferred_element_type=jnp.float32) + b[:, 0:HID]
    k_ref[...] = jnp.dot(h, wk_ref[...], preferred_element_type=jnp.float32) + b[:, HID:2 * HID]
    v_ref[...] = jnp.dot(h, wv_ref[...], preferred_element_type=jnp.float32) + b[:, 2 * HID:3 * HID]
    s_ref[...] = jnp.dot(h, ws_ref[...], preferred_element_type=jnp.float32) + b[:, 3 * HID:4 * HID]


def _qkvs_proj(h, wq, wk, wv, ws, bcat):
    grid = (N // TILE_N,)
    wspec = pl.BlockSpec((HID, HID), lambda i: (0, 0))
    nspec = pl.BlockSpec((TILE_N, HID), lambda i: (i, 0))
    return pl.pallas_call(
        _qkvs_body,
        grid=grid,
        in_specs=[nspec, wspec, wspec, wspec, wspec,
                  pl.BlockSpec((1, 4 * HID), lambda i: (0, 0))],
        out_specs=[nspec, nspec, nspec, nspec],
        out_shape=[jax.ShapeDtypeStruct((N, HID), jnp.float32)] * 4,
    )(h, wq, wk, wv, ws, bcat)


def _final_body(h_ref, w_ref, b_ref, out_ref):
    out_ref[...] = (jnp.dot(h_ref[...], w_ref[...],
                            preferred_element_type=jnp.float32) + b_ref[...])


def _final_proj(h, w, b2d):
    grid = (N // TILE_N,)
    return pl.pallas_call(
        _final_body,
        grid=grid,
        in_specs=[
            pl.BlockSpec((TILE_N, HID), lambda i: (i, 0)),
            pl.BlockSpec((HID, HID), lambda i: (0, 0)),
            pl.BlockSpec((1, HID), lambda i: (0, 0)),
        ],
        out_specs=pl.BlockSpec((TILE_N, HID), lambda i: (i, 0)),
        out_shape=jax.ShapeDtypeStruct((N, HID), jnp.float32),
    )(h, w, b2d)


def _post_body(pa_ref, pd_ref, rep_ref, skip_ref, h_ref, lw_ref, lb_ref,
               out_ref):
    pa = pa_ref[...]
    pd = pd_ref[...]
    aggnum = pa[0] + pa[1]
    den = pd[0, :, 0:HEADS] + pd[1, :, 0:HEADS]
    denrep = jnp.dot(den, rep_ref[...], preferred_element_type=jnp.float32)
    agg = aggnum / (denrep + 1e-16)
    t = _ssp(agg + skip_ref[...])
    out_ref[...] = (h_ref[...]
                    + jnp.dot(t, lw_ref[...], preferred_element_type=jnp.float32)
                    + lb_ref[...])


def _post(pagg, pden, rep, skip, h, lin_w, lin_b):
    grid = (N // TILE_N,)
    return pl.pallas_call(
        _post_body,
        grid=grid,
        in_specs=[
            pl.BlockSpec((2, TILE_N, HID), lambda i: (0, i, 0)),
            pl.BlockSpec((2, TILE_N, DEN_W), lambda i: (0, i, 0)),
            pl.BlockSpec((HEADS, HID), lambda i: (0, 0)),
            pl.BlockSpec((TILE_N, HID), lambda i: (i, 0)),
            pl.BlockSpec((TILE_N, HID), lambda i: (i, 0)),
            pl.BlockSpec((HID, HID), lambda i: (0, 0)),
            pl.BlockSpec((1, HID), lambda i: (0, 0)),
        ],
        out_specs=pl.BlockSpec((TILE_N, HID), lambda i: (i, 0)),
        out_shape=jax.ShapeDtypeStruct((N, HID), jnp.float32),
    )(pagg, pden, rep, skip, h, lin_w, lin_b)


def _attn_sc_body(q_hbm, k_hbm, v_hbm, ep_hbm, src_hbm, dst_hbm,
                  agg_hbm, den_hbm,
                  src_v0, src_v1, dst_v0, dst_v1, dst_s0, dst_s1,
                  q_r0, q_r1, k_r0, k_r1, v_r0, v_r1, e_r0, e_r1,
                  w_r, ea_b,
                  agg_tbl, den_tbl,
                  sem_i0, sem_i1, sem_g0, sem_g1, sem_s):
    cid = lax.axis_index("c")
    sid = lax.axis_index("s")
    w = cid * 16 + sid

    src_v = (src_v0, src_v1)
    dst_v = (dst_v0, dst_v1)
    dst_s = (dst_s0, dst_s1)
    q_r = (q_r0, q_r1)
    k_r = (k_r0, k_r1)
    v_r = (v_r0, v_r1)
    e_r = (e_r0, e_r1)
    sem_i = (sem_i0, sem_i1)
    sem_g = (sem_g0, sem_g1)

    lanes = lax.iota(jnp.int32, 16)
    zf = jnp.zeros((16,), jnp.float32)

    def _zero_bufs(r, carry):
        rows = jnp.zeros((16,), jnp.int32) + r
        for c in range(HID // 16):
            plsc.store_scatter(w_r, [rows, c * 16 + lanes], zf)
        plsc.store_scatter(ea_b, [rows, lanes], zf)
        return carry
    lax.fori_loop(0, BB, _zero_bufs, 0)

    r0 = sid * ROWS_PER_TILE
    for off in range(0, ROWS_PER_TILE, BB):
        n = min(BB, ROWS_PER_TILE - off)
        pltpu.sync_copy(w_r.at[pl.ds(0, n)], agg_tbl.at[pl.ds(r0 + off, n)])
        pltpu.sync_copy(ea_b.at[pl.ds(0, n)], den_tbl.at[pl.ds(r0 + off, n)])
    plsc.subcore_barrier()

    def _issue_idx(slot, p):
        base = (w + NW * slot) * BB
        pltpu.async_copy(src_hbm.at[pl.ds(base, BB)], src_v[p], sem_i[p])
        pltpu.async_copy(dst_hbm.at[pl.ds(base, BB)], dst_v[p], sem_i[p])

    def _wait_idx(slot, p):
        base = (w + NW * slot) * BB
        pltpu.make_async_copy(src_hbm.at[pl.ds(base, BB)], src_v[p],
                              sem_i[p]).wait()
        pltpu.make_async_copy(dst_hbm.at[pl.ds(base, BB)], dst_v[p],
                              sem_i[p]).wait()

    def _issue_gathers(slot, p):
        base = (w + NW * slot) * BB
        pltpu.async_copy(q_hbm.at[dst_v[p]], q_r[p], sem_g[p])
        pltpu.async_copy(k_hbm.at[src_v[p]], k_r[p], sem_g[p])
        pltpu.async_copy(v_hbm.at[src_v[p]], v_r[p], sem_g[p])
        pltpu.async_copy(ep_hbm.at[pl.ds(base, BB)], e_r[p], sem_g[p])

    def _wait_gathers(slot, p):
        base = (w + NW * slot) * BB
        pltpu.make_async_copy(q_hbm.at[dst_v[p]], q_r[p], sem_g[p]).wait()
        pltpu.make_async_copy(k_hbm.at[src_v[p]], k_r[p], sem_g[p]).wait()
        pltpu.make_async_copy(v_hbm.at[src_v[p]], v_r[p], sem_g[p]).wait()
        pltpu.make_async_copy(ep_hbm.at[pl.ds(base, BB)], e_r[p],
                              sem_g[p]).wait()

    def _copy_dst(p):
        for t in range(BB // 16):
            dst_s[p][pl.ds(t * 16, 16)] = dst_v[p][pl.ds(t * 16, 16)]

    def _issue_scatter(p):
        pltpu.async_copy(w_r, agg_tbl.at[dst_s[p]], sem_s, add=True)
        pltpu.async_copy(ea_b, den_tbl.at[dst_s[p]], sem_s, add=True)

    def _wait_scatter(p):
        pltpu.make_async_copy(w_r, agg_tbl.at[dst_s[p]], sem_s).wait()
        pltpu.make_async_copy(ea_b, den_tbl.at[dst_s[p]], sem_s).wait()

    def _compute(p):
        def _edge(e, carry2):
            earow = zf
            for h in range(HEADS):
                sl = pl.ds(h * DH, DH)
                qs = q_r[p][e, sl]
                ks = k_r[p][e, sl]
                es = e_r[p][e, sl]
                s = jnp.sum(qs * (ks + es))
                ea = jnp.exp(jnp.full((16,), s, jnp.float32) * INV_SQRT_DH)
                vs = v_r[p][e, sl]
                w_r[e, sl] = (vs + es) * ea
                earow = jnp.where(lanes == h, ea, earow)
            ea_b[e, pl.ds(0, DEN_W)] = earow
            return carry2
        lax.fori_loop(0, BB, _edge, 0)

    def _valid(slot):
        return (w + NW * slot) < NB_TOT

    _issue_idx(0, 0)
    _wait_idx(0, 0)
    _issue_gathers(0, 0)

    @pl.when(_valid(1))
    def _():
        _issue_idx(1, 1)

    def _slot(j, p):
        jn1 = j + 1
        jn2 = j + 2

        @pl.when(_valid(jn1))
        def _():
            _wait_idx(jn1, 1 - p)
            _issue_gathers(jn1, 1 - p)

        @pl.when(_valid(j))
        def _():
            _wait_gathers(j, p)
            _copy_dst(p)

        @pl.when(_valid(jn2))
        def _():
            _issue_idx(jn2, p)

        @pl.when((j >= 1) & _valid(j - 1))
        def _():
            _wait_scatter(1 - p)

        @pl.when(_valid(j))
        def _():
            _compute(p)
            _issue_scatter(p)

    def _pair(jj, carry):
        j = 2 * jj
        _slot(j, 0)
        _slot(j + 1, 1)
        return carry
    lax.fori_loop(0, (NB_PER_W + 1) // 2, _pair, 0)

    plsc.subcore_barrier()
    for off in range(0, ROWS_PER_TILE, BB):
        n = min(BB, ROWS_PER_TILE - off)
        pltpu.sync_copy(agg_tbl.at[pl.ds(r0 + off, n)], w_r.at[pl.ds(0, n)])
        pltpu.sync_copy(w_r.at[pl.ds(0, n)],
                        agg_hbm.at[cid, pl.ds(r0 + off, n)])
        pltpu.sync_copy(den_tbl.at[pl.ds(r0 + off, n)], ea_b.at[pl.ds(0, n)])
        pltpu.sync_copy(ea_b.at[pl.ds(0, n)],
                        den_hbm.at[cid, pl.ds(r0 + off, n)])


@functools.partial(
    pl.kernel,
    mesh=plsc.VectorSubcoreMesh(core_axis_name="c", subcore_axis_name="s"),
    out_type=(jax.ShapeDtypeStruct((2, N_PAD, HID), jnp.float32),
              jax.ShapeDtypeStruct((2, N_PAD, DEN_W), jnp.float32)),
    compiler_params=pltpu.CompilerParams(use_tc_tiling_on_sc=False,
                                         needs_layout_passes=False),
    scratch_types=(
        [pltpu.VMEM((BB,), jnp.int32)] * 6
        + [pltpu.VMEM((BB, HID), jnp.float32)] * 9
        + [pltpu.VMEM((BB, DEN_W), jnp.float32)] * 1
        + [pltpu.VMEM_SHARED((N_PAD, HID), jnp.float32),
           pltpu.VMEM_SHARED((N_PAD, DEN_W), jnp.float32)]
        + [pltpu.SemaphoreType.DMA] * 5
    ),
)
def _attn_sc(q_hbm, k_hbm, v_hbm, ep_hbm, src_hbm, dst_hbm, agg_hbm, den_hbm,
             *rest):
    _attn_sc_body(q_hbm, k_hbm, v_hbm, ep_hbm, src_hbm, dst_hbm,
                  agg_hbm, den_hbm, *rest)


def kernel(x, edge_index, edge_weight, edge_attr, mlp_w1, mlp_b1, mlp_w2,
           mlp_b2, wq, bq, wk, bk, wv, bv, we, be, wskip, bskip, lin_w, lin_b,
           out_w, out_b):
    src = edge_index[0]
    dst = edge_index[1]
    ew2d = edge_weight.reshape(E, 1)
    rep = jnp.repeat(jnp.eye(HEADS, dtype=jnp.float32), DH, axis=1)
    h = x
    for l in range(L):
        ep = _edge_proj(edge_attr, ew2d, mlp_w1[l], mlp_b1[l].reshape(1, FIL),
                        mlp_w2[l], mlp_b2[l].reshape(1, FIL), we[l],
                        be[l].reshape(1, HID))
        bcat = jnp.concatenate([bq[l], bk[l], bv[l], bskip[l]]).reshape(1, 4 * HID)
        q, k, v, skip = _qkvs_proj(h, wq[l], wk[l], wv[l], wskip[l], bcat)
        pagg, pden = _attn_sc(q, k, v, ep, src, dst)
        h = _post(pagg, pden, rep, skip, h, lin_w[l], lin_b[l].reshape(1, HID))
    return _final_proj(h, out_w, out_b.reshape(1, HID))
